# baseline (speedup 1.0000x reference)
import functools

import jax
import jax.numpy as jnp
from jax import lax
from jax.experimental import pallas as pl
from jax.experimental.pallas import tpu as pltpu
from jax.experimental.pallas import tpu_sc as plsc

N = 1024
F = 64
K = 16
L = 16
NC = 2
NS = 16
NW = NC * NS
JW = N // NW
FU = 16
NCHUNK = N // L


def _sc_body(xT_hbm, adjT_hbm, x_hbm, out_hbm, xT_v, row_v, buf_v):
    wid = lax.axis_index("s") * NC + lax.axis_index("c")
    pltpu.sync_copy(xT_hbm, xT_v)

    rows_idx = lax.iota(jnp.int32, L) + 1
    neg_inf = jnp.full((L,), -jnp.inf, jnp.float32)

    def j_body(jj, carry):
        j = wid * JW + jj
        pltpu.sync_copy(adjT_hbm.at[j], row_v)
        pltpu.sync_copy(x_hbm.at[j], buf_v.at[0])

        for fg in range(F // FU):
            def chunk_body(c, runs):
                base = c * L
                a = row_v[pl.ds(base, L)]
                new_runs = []
                for u in range(FU):
                    xv = xT_v[fg * FU + u, pl.ds(base, L)]
                    p, _ = plsc.sort_key_val(a * xv, a * xv)
                    m = jnp.maximum(runs[u], p)
                    r, _ = plsc.sort_key_val(m, m, descending=True)
                    new_runs.append(r)
                return tuple(new_runs)

            runs = plsc.parallel_loop(
                0, NCHUNK, 1, unroll=4,
                carry=tuple(neg_inf for _ in range(FU)),
            )(chunk_body)
            for u in range(FU):
                cols = jnp.full((L,), fg * FU + u, jnp.int32)
                plsc.store_scatter(buf_v, [rows_idx, cols], runs[u])

        pltpu.sync_copy(buf_v, out_hbm.at[j])
        return carry

    lax.fori_loop(0, JW, j_body, 0)


def kernel(x, adj):
    xT = jnp.transpose(x)
    adjT = jnp.transpose(adj)

    mesh = plsc.VectorSubcoreMesh(core_axis_name="c", subcore_axis_name="s")
    run = pl.kernel(
        _sc_body,
        out_type=jax.ShapeDtypeStruct((N, K + 1, F), jnp.float32),
        mesh=mesh,
        compiler_params=pltpu.CompilerParams(needs_layout_passes=False),
        scratch_types=[
            pltpu.VMEM((F, N), jnp.float32),
            pltpu.VMEM((N,), jnp.float32),
            pltpu.VMEM((K + 1, F), jnp.float32),
        ],
    )
    return run(xT, adjT, x)


# speedup vs baseline: 1.0701x; 1.0701x over previous
import functools

import jax
import jax.numpy as jnp
from jax import lax
from jax.experimental import pallas as pl
from jax.experimental.pallas import tpu as pltpu
from jax.experimental.pallas import tpu_sc as plsc

N = 1024
F = 64
K = 16
L = 16
NC = 2
NS = 16
NW = NC * NS
JW = N // NW
FU = 8
NCHUNK = N // L


def _sc_body(xT_hbm, adjT_hbm, x_hbm, out_hbm, xT_v, row_v, buf_v):
    wid = lax.axis_index("s") * NC + lax.axis_index("c")
    pltpu.sync_copy(xT_hbm, xT_v)

    rows_idx = lax.iota(jnp.int32, L) + 1
    neg_inf = jnp.full((L,), -jnp.inf, jnp.float32)

    def j_body(jj, carry):
        j = wid * JW + jj
        pltpu.sync_copy(adjT_hbm.at[j], row_v)
        pltpu.sync_copy(x_hbm.at[j], buf_v.at[0])

        for fg in range(F // FU):
            def chunk_body(c, runs):
                base = c * L
                a = row_v[pl.ds(base, L)]
                new_runs = []
                for u in range(FU):
                    xv = xT_v[fg * FU + u, pl.ds(base, L)]
                    p, _ = plsc.sort_key_val(a * xv, a * xv)
                    m = jnp.maximum(runs[u], p)
                    r, _ = plsc.sort_key_val(m, m, descending=True)
                    new_runs.append(r)
                return tuple(new_runs)

            runs = plsc.parallel_loop(
                0, NCHUNK, 1, unroll=4,
                carry=tuple(neg_inf for _ in range(FU)),
            )(chunk_body)
            for u in range(FU):
                cols = jnp.full((L,), fg * FU + u, jnp.int32)
                plsc.store_scatter(buf_v, [rows_idx, cols], runs[u])

        pltpu.sync_copy(buf_v, out_hbm.at[j])
        return carry

    lax.fori_loop(0, JW, j_body, 0)


def kernel(x, adj):
    xT = jnp.transpose(x)
    adjT = jnp.transpose(adj)

    mesh = plsc.VectorSubcoreMesh(core_axis_name="c", subcore_axis_name="s")
    run = pl.kernel(
        _sc_body,
        out_type=jax.ShapeDtypeStruct((N, K + 1, F), jnp.float32),
        mesh=mesh,
        compiler_params=pltpu.CompilerParams(needs_layout_passes=False),
        scratch_types=[
            pltpu.VMEM((F, N), jnp.float32),
            pltpu.VMEM((N,), jnp.float32),
            pltpu.VMEM((K + 1, F), jnp.float32),
        ],
    )
    return run(xT, adjT, x)


# speedup vs baseline: 1.1493x; 1.0740x over previous
import functools

import jax
import jax.numpy as jnp
from jax import lax
from jax.experimental import pallas as pl
from jax.experimental.pallas import tpu as pltpu
from jax.experimental.pallas import tpu_sc as plsc

N = 1024
F = 64
K = 16
L = 16
NC = 2
NS = 16
NW = NC * NS
JW = N // NW
FU = 8
NCHUNK = N // L


def _sc_body(xT_hbm, adjT_hbm, x_hbm, out_hbm, xT_v, row_v, buf_v,
             sem_row, sem_out):
    wid = lax.axis_index("s") * NC + lax.axis_index("c")
    j0 = wid * JW
    pltpu.sync_copy(xT_hbm, xT_v)
    pltpu.sync_copy(adjT_hbm.at[j0], row_v.at[0])

    rows_idx = lax.iota(jnp.int32, L) + 1
    neg_inf = jnp.full((L,), -jnp.inf, jnp.float32)

    def j_body(jj, carry):
        j = j0 + jj
        slot = jj % 2
        nslot = (jj + 1) % 2
        j_next = jnp.minimum(j + 1, j0 + JW - 1)
        pref = pltpu.async_copy(adjT_hbm.at[j_next], row_v.at[nslot], sem_row)

        @pl.when(jj >= 1)
        def _():
            pltpu.make_async_copy(
                buf_v.at[nslot], out_hbm.at[jnp.maximum(j - 1, j0)], sem_out
            ).wait()

        pltpu.sync_copy(x_hbm.at[j], buf_v.at[slot, 0])

        for fg in range(F // FU):
            def chunk_body(c, runs):
                base = c * L
                a = row_v[slot, pl.ds(base, L)]
                new_runs = []
                for u in range(FU):
                    xv = xT_v[fg * FU + u, pl.ds(base, L)]
                    p, _ = plsc.sort_key_val(a * xv, a * xv)
                    m = jnp.maximum(runs[u], p)
                    r, _ = plsc.sort_key_val(m, m, descending=True)
                    new_runs.append(r)
                return tuple(new_runs)

            runs = plsc.parallel_loop(
                0, NCHUNK, 1, unroll=4,
                carry=tuple(neg_inf for _ in range(FU)),
            )(chunk_body)
            for u in range(FU):
                cols = jnp.full((L,), fg * FU + u, jnp.int32)
                plsc.store_scatter(buf_v.at[slot], [rows_idx, cols], runs[u])

        pltpu.async_copy(buf_v.at[slot], out_hbm.at[j], sem_out)
        pref.wait()
        return carry

    lax.fori_loop(0, JW, j_body, 0)
    pltpu.make_async_copy(
        buf_v.at[(JW - 1) % 2], out_hbm.at[j0 + JW - 1], sem_out
    ).wait()


def kernel(x, adj):
    xT = jnp.transpose(x)
    adjT = jnp.transpose(adj)

    mesh = plsc.VectorSubcoreMesh(core_axis_name="c", subcore_axis_name="s")
    run = pl.kernel(
        _sc_body,
        out_type=jax.ShapeDtypeStruct((N, K + 1, F), jnp.float32),
        mesh=mesh,
        compiler_params=pltpu.CompilerParams(needs_layout_passes=False),
        scratch_types=[
            pltpu.VMEM((F, N), jnp.float32),
            pltpu.VMEM((2, N), jnp.float32),
            pltpu.VMEM((2, K + 1, F), jnp.float32),
            pltpu.SemaphoreType.DMA,
            pltpu.SemaphoreType.DMA,
        ],
    )
    return run(xT, adjT, x)
